# baseline (speedup 1.0000x reference)
import functools

import jax
import jax.numpy as jnp
from jax import lax
from jax.experimental import pallas as pl
from jax.experimental.pallas import tpu as pltpu, tpu_sc as plsc

_info = plsc.get_sparse_core_info()
_NC = _info.num_cores
_NS = _info.num_subcores
_NW = _NC * _NS


def _make_gather(B, D):
  assert B % (8 * _NW) == 0 and D % _info.num_lanes == 0
  b_per_w = B // _NW
  mesh = plsc.VectorSubcoreMesh(core_axis_name="c", subcore_axis_name="s")

  @functools.partial(
      pl.kernel,
      mesh=mesh,
      out_type=jax.ShapeDtypeStruct((B, D), jnp.float32),
      scratch_types=[
          pltpu.VMEM((b_per_w,), jnp.int32),
          pltpu.VMEM((b_per_w, D), jnp.float32),
          pltpu.SemaphoreType.DMA,
      ],
      compiler_params=pltpu.CompilerParams(use_tc_tiling_on_sc=False),
  )
  def gather_kernel(table_hbm, idx_hbm, out_hbm, idx_v, rows_v, sem):
    wid = lax.axis_index("s") * _NC + lax.axis_index("c")
    base = wid * b_per_w
    pltpu.sync_copy(idx_hbm.at[pl.ds(base, b_per_w)], idx_v)
    pltpu.async_copy(table_hbm.at[idx_v], rows_v, sem).wait()
    pltpu.sync_copy(rows_v, out_hbm.at[pl.ds(base, b_per_w)])

  return gather_kernel


@jax.jit
def kernel(users, U_g):
  flat = users.reshape(-1).astype(jnp.int32)
  out = _make_gather(flat.shape[0], U_g.shape[1])(U_g, flat)
  return out.reshape(tuple(users.shape) + (U_g.shape[1],))


# speedup vs baseline: 1.6436x; 1.6436x over previous
import functools

import jax
import jax.numpy as jnp
from jax import lax
from jax.experimental import pallas as pl
from jax.experimental.pallas import tpu as pltpu, tpu_sc as plsc

_info = plsc.get_sparse_core_info()
_NC = _info.num_cores
_NS = _info.num_subcores
_NW = _NC * _NS


def _make_gather(B, D):
  b_per_w = B // _NW
  mesh = plsc.VectorSubcoreMesh(core_axis_name="c", subcore_axis_name="s")

  @functools.partial(
      pl.kernel,
      mesh=mesh,
      out_type=jax.ShapeDtypeStruct((B, D), jnp.float32),
      scratch_types=[
          pltpu.VMEM((b_per_w,), jnp.int32),
          pltpu.VMEM((b_per_w, D), jnp.float32),
          pltpu.SemaphoreType.DMA,
          pltpu.SemaphoreType.DMA,
      ],
  )
  def gather_kernel(table_hbm, idx_hbm, out_hbm, idx_v, rows_v, sem, sem2):
    wid = lax.axis_index("s") * _NC + lax.axis_index("c")
    base = wid * b_per_w
    pltpu.sync_copy(idx_hbm.at[pl.ds(base, b_per_w)], idx_v)

    def body(g, _):
      vec = idx_v[pl.ds(g * 16, 16)]
      copies = []
      for k in range(16):
        r = vec[k]
        copies.append(pltpu.async_copy(table_hbm.at[r], rows_v.at[g * 16 + k], sem))
      for c in copies:
        c.wait()
      return 0

    lax.fori_loop(0, b_per_w // 16, body, 0)
    pltpu.sync_copy(rows_v, out_hbm.at[pl.ds(base, b_per_w)])

  return gather_kernel


@jax.jit
def kernel(users, U_g):
  flat = users.reshape(-1).astype(jnp.int32)
  out = _make_gather(flat.shape[0], U_g.shape[1])(U_g, flat)
  return out.reshape(tuple(users.shape) + (U_g.shape[1],))


# speedup vs baseline: 1.7276x; 1.0511x over previous
import functools

import jax
import jax.numpy as jnp
from jax import lax
from jax.experimental import pallas as pl
from jax.experimental.pallas import tpu as pltpu, tpu_sc as plsc

_info = plsc.get_sparse_core_info()
_NC = _info.num_cores
_NS = _info.num_subcores
_NW = _NC * _NS

_G = 16


def _make_gather(B, D):
  b_per_w = B // _NW
  n_groups = b_per_w // _G
  mesh = plsc.VectorSubcoreMesh(core_axis_name="c", subcore_axis_name="s")

  @functools.partial(
      pl.kernel,
      mesh=mesh,
      out_type=jax.ShapeDtypeStruct((B, D), jnp.float32),
      scratch_types=[
          pltpu.VMEM((b_per_w,), jnp.int32),
          pltpu.VMEM((b_per_w, D), jnp.float32),
          pltpu.SemaphoreType.DMA,
      ],
  )
  def gather_kernel(table_hbm, idx_hbm, out_hbm, idx_v, rows_v, sem):
    wid = lax.axis_index("s") * _NC + lax.axis_index("c")
    base = wid * b_per_w
    pltpu.sync_copy(idx_hbm.at[pl.ds(base, b_per_w)], idx_v)

    def fire(g, _):
      vec = idx_v[pl.ds(g * _G, _G)]
      for k in range(_G):
        pltpu.async_copy(table_hbm.at[vec[k]], rows_v.at[g * _G + k], sem)
      return 0

    lax.fori_loop(0, n_groups, fire, 0)

    def drain(g, _):
      for k in range(_G):
        pltpu.make_async_copy(table_hbm.at[0], rows_v.at[0], sem).wait()
      return 0

    lax.fori_loop(0, n_groups, drain, 0)
    pltpu.sync_copy(rows_v, out_hbm.at[pl.ds(base, b_per_w)])

  return gather_kernel


@jax.jit
def kernel(users, U_g):
  flat = users.reshape(-1).astype(jnp.int32)
  out = _make_gather(flat.shape[0], U_g.shape[1])(U_g, flat)
  return out.reshape(tuple(users.shape) + (U_g.shape[1],))
